# baseline (speedup 1.0000x reference)
import functools

import jax
import jax.numpy as jnp
from jax.experimental import pallas as pl
from jax.experimental.pallas import tpu as pltpu

B = 8
C = 3
IMG = 224
P = 16
D = 768
DFF = 3072
E = 8
G = IMG // P
S = G * G
N = B * S
PP = P * P
TF = 768


def _dt(a, b):
    return jax.lax.dot_general(a, b, (((1,), (1,)), ((), ())),
                               preferred_element_type=jnp.float32)


def _embed_router_body(x_ref, pwh_ref, pwl_ref, pb_ref, rw_ref,
                       flat_ref, xn_ref, comb_ref, mh_ref, ml_ref):
    @pl.when(pl.program_id(0) == 0)
    def _router_mat():
        rw = rw_ref[...]
        rwh = rw.astype(jnp.bfloat16)
        rwl = (rw - rwh.astype(jnp.float32)).astype(jnp.bfloat16)
        d = lambda p, q: jax.lax.dot_general(
            p, q, (((1,), (0,)), ((), ())),
            preferred_element_type=jnp.float32)
        m = d(rwh, pwh_ref[...]) + (d(rwh, pwl_ref[...])
                                    + d(rwl, pwh_ref[...]))
        mh = m.astype(jnp.bfloat16)
        mh_ref[...] = mh
        ml_ref[...] = (m - mh.astype(jnp.float32)).astype(jnp.bfloat16)

    x = x_ref[0]
    xh = x.astype(jnp.bfloat16)
    xl = (x - xh.astype(jnp.float32)).astype(jnp.bfloat16)
    flat = None
    logits = None
    for c in range(C):
        sl = slice(c * PP, (c + 1) * PP)
        part = _dt(xh[c], pwh_ref[:, sl])
        flat = part if flat is None else flat + part
        lg = (_dt(xh[c], mh_ref[:, sl]) + _dt(xh[c], ml_ref[:, sl])
              + _dt(xl[c], mh_ref[:, sl]))
        logits = lg if logits is None else logits + lg
    flat = flat + pb_ref[...]
    flat_ref[0] = flat
    mean = jnp.mean(flat, axis=1, keepdims=True)
    var = jnp.mean((flat - mean) ** 2, axis=1, keepdims=True)
    xn_ref[0] = (flat - mean) * jax.lax.rsqrt(var + 1e-5)

    logits = logits + _dt(pb_ref[...], rw_ref[...])
    idx = jax.lax.broadcasted_iota(jnp.int32, logits.shape, 1)
    v1 = jnp.max(logits, axis=1, keepdims=True)
    i1 = jnp.min(jnp.where(logits == v1, idx, E), axis=1, keepdims=True)
    rest = jnp.where(idx == i1, -jnp.inf, logits)
    v2 = jnp.max(rest, axis=1, keepdims=True)
    i2 = jnp.min(jnp.where(rest == v2, idx, E), axis=1, keepdims=True)
    w1 = 1.0 / (1.0 + jnp.exp(v2 - v1))
    w2 = 1.0 - w1
    comb_ref[0] = (jnp.where(idx == i1, w1, 0.0)
                   + jnp.where(idx == i2, w2, 0.0))


def _expert_body(xn_ref, lng_ref, lnb_ref, fc1_ref, f1b_ref, fc2_ref,
                 f2b_ref, comb_ref, flat_ref, out_ref, xne_ref):
    e = pl.program_id(0)
    f = pl.program_id(1)

    @pl.when(jnp.logical_and(e == 0, f == 0))
    def _init():
        out_ref[...] = flat_ref[...]

    @pl.when(f == 0)
    def _scale_shift():
        xne_ref[...] = (xn_ref[...] * lng_ref[0]
                        + lnb_ref[0]).astype(jnp.bfloat16)

    eidx = jax.lax.broadcasted_iota(jnp.int32, (N, E), 1)
    c = jnp.sum(jnp.where(eidx == e, comb_ref[...], 0.0), axis=1,
                keepdims=True)

    w1 = fc1_ref[0].astype(jnp.bfloat16)
    h = jax.lax.dot_general(xne_ref[...], w1, (((1,), (1,)), ((), ())),
                            preferred_element_type=jnp.float32) + f1b_ref[0]
    h = jax.nn.gelu(h)
    w2 = fc2_ref[0].astype(jnp.bfloat16)
    eo = jax.lax.dot_general(h.astype(jnp.bfloat16), w2,
                             (((1,), (1,)), ((), ())),
                             preferred_element_type=jnp.float32)

    @pl.when(f == 0)
    def _bias():
        out_ref[...] += c * f2b_ref[0]

    out_ref[...] += c * eo


@jax.jit
def kernel(images, proj_w, proj_b, router_w, ln_g, ln_b,
           fc1_w, fc1_b, fc2_w, fc2_b):
    x4 = images.reshape(B, C, G, P, G, P).transpose(0, 1, 2, 4, 3, 5)
    x4 = x4.reshape(B, C, S, PP)
    pw_hi = proj_w.astype(jnp.bfloat16)
    pw_lo = (proj_w - pw_hi.astype(jnp.float32)).astype(jnp.bfloat16)

    flat3, xn3, comb3 = pl.pallas_call(
        _embed_router_body,
        grid=(B,),
        in_specs=[
            pl.BlockSpec((1, C, S, PP), lambda b: (b, 0, 0, 0)),
            pl.BlockSpec((D, C * PP), lambda b: (0, 0)),
            pl.BlockSpec((D, C * PP), lambda b: (0, 0)),
            pl.BlockSpec((1, D), lambda b: (0, 0)),
            pl.BlockSpec((E, D), lambda b: (0, 0)),
        ],
        out_specs=[
            pl.BlockSpec((1, S, D), lambda b: (b, 0, 0)),
            pl.BlockSpec((1, S, D), lambda b: (b, 0, 0)),
            pl.BlockSpec((1, S, E), lambda b: (b, 0, 0)),
        ],
        out_shape=[
            jax.ShapeDtypeStruct((B, S, D), jnp.float32),
            jax.ShapeDtypeStruct((B, S, D), jnp.float32),
            jax.ShapeDtypeStruct((B, S, E), jnp.float32),
        ],
        scratch_shapes=[pltpu.VMEM((E, C * PP), jnp.bfloat16),
                        pltpu.VMEM((E, C * PP), jnp.bfloat16)],
    )(x4, pw_hi, pw_lo, proj_b.reshape(1, D), router_w)

    flat = flat3.reshape(N, D)
    xn = xn3.reshape(N, D)
    comb = comb3.reshape(N, E)

    out = pl.pallas_call(
        _expert_body,
        grid=(E, DFF // TF),
        in_specs=[
            pl.BlockSpec((N, D), lambda e, f: (0, 0)),
            pl.BlockSpec((1, 1, D), lambda e, f: (e, 0, 0)),
            pl.BlockSpec((1, 1, D), lambda e, f: (e, 0, 0)),
            pl.BlockSpec((1, TF, D), lambda e, f: (e, f, 0)),
            pl.BlockSpec((1, 1, TF), lambda e, f: (e, 0, f)),
            pl.BlockSpec((1, D, TF), lambda e, f: (e, 0, f)),
            pl.BlockSpec((1, 1, D), lambda e, f: (e, 0, 0)),
            pl.BlockSpec((N, E), lambda e, f: (0, 0)),
            pl.BlockSpec((N, D), lambda e, f: (0, 0)),
        ],
        out_specs=pl.BlockSpec((N, D), lambda e, f: (0, 0)),
        out_shape=jax.ShapeDtypeStruct((N, D), jnp.float32),
        scratch_shapes=[pltpu.VMEM((N, D), jnp.bfloat16)],
    )(xn, ln_g.reshape(E, 1, D), ln_b.reshape(E, 1, D), fc1_w,
      fc1_b.reshape(E, 1, DFF), fc2_w, fc2_b.reshape(E, 1, D), comb, flat)

    return out.reshape(B, S, D)


# speedup vs baseline: 1.0156x; 1.0156x over previous
import functools

import jax
import jax.numpy as jnp
from jax.experimental import pallas as pl
from jax.experimental.pallas import tpu as pltpu

B = 8
C = 3
IMG = 224
P = 16
D = 768
DFF = 3072
E = 8
G = IMG // P
S = G * G
N = B * S
PP = P * P
K = C * PP
TN = 224
TF = 768


def _split(a):
    hi = a.astype(jnp.bfloat16)
    lo = (a - hi.astype(jnp.float32)).astype(jnp.bfloat16)
    return hi, lo


def _dt(a, b):
    return jax.lax.dot_general(a, b, (((1,), (1,)), ((), ())),
                               preferred_element_type=jnp.float32)


def _embed_router_body(x_ref, pw_ref, pb_ref, rw_ref,
                       flat_ref, xn_ref, comb_ref,
                       pwh_ref, mh_ref, ml_ref):
    @pl.when(pl.program_id(0) == 0)
    def _prep():
        pwh, pwl = _split(pw_ref[...])
        pwh_ref[...] = pwh
        rwh, rwl = _split(rw_ref[...])
        d = lambda p, q: jax.lax.dot_general(
            p, q, (((1,), (0,)), ((), ())),
            preferred_element_type=jnp.float32)
        m = d(rwh, pwh) + (d(rwh, pwl) + d(rwl, pwh))
        mh, ml = _split(m)
        mh_ref[...] = mh
        ml_ref[...] = ml

    x = x_ref[...]
    xh, xl = _split(x)
    flat = _dt(xh, pwh_ref[...]) + pb_ref[...]
    flat_ref[...] = flat
    mean = jnp.mean(flat, axis=1, keepdims=True)
    var = jnp.mean((flat - mean) ** 2, axis=1, keepdims=True)
    xn_ref[...] = (flat - mean) * jax.lax.rsqrt(var + 1e-5)

    logits = (_dt(xh, mh_ref[...]) + _dt(xh, ml_ref[...])
              + _dt(xl, mh_ref[...]))
    logits = logits + _dt(pb_ref[...], rw_ref[...])
    idx = jax.lax.broadcasted_iota(jnp.int32, logits.shape, 1)
    v1 = jnp.max(logits, axis=1, keepdims=True)
    i1 = jnp.min(jnp.where(logits == v1, idx, E), axis=1, keepdims=True)
    rest = jnp.where(idx == i1, -jnp.inf, logits)
    v2 = jnp.max(rest, axis=1, keepdims=True)
    i2 = jnp.min(jnp.where(rest == v2, idx, E), axis=1, keepdims=True)
    w1 = 1.0 / (1.0 + jnp.exp(v2 - v1))
    w2 = 1.0 - w1
    comb_ref[...] = (jnp.where(idx == i1, w1, 0.0)
                     + jnp.where(idx == i2, w2, 0.0))


def _expert_body(xn_ref, lng_ref, lnb_ref, fc1_ref, f1b_ref, fc2_ref,
                 f2b_ref, comb_ref, flat_ref, out_ref, xne_ref):
    e = pl.program_id(0)
    f = pl.program_id(1)

    @pl.when(jnp.logical_and(e == 0, f == 0))
    def _init():
        out_ref[...] = flat_ref[...]

    @pl.when(f == 0)
    def _scale_shift():
        xne_ref[...] = (xn_ref[...] * lng_ref[0]
                        + lnb_ref[0]).astype(jnp.bfloat16)

    eidx = jax.lax.broadcasted_iota(jnp.int32, (N, E), 1)
    c = jnp.sum(jnp.where(eidx == e, comb_ref[...], 0.0), axis=1,
                keepdims=True)

    w1 = fc1_ref[0].astype(jnp.bfloat16)
    h = jax.lax.dot_general(xne_ref[...], w1, (((1,), (1,)), ((), ())),
                            preferred_element_type=jnp.float32) + f1b_ref[0]
    h = jax.nn.gelu(h)
    w2 = fc2_ref[0].astype(jnp.bfloat16)
    eo = jax.lax.dot_general(h.astype(jnp.bfloat16), w2,
                             (((1,), (1,)), ((), ())),
                             preferred_element_type=jnp.float32)

    @pl.when(f == 0)
    def _bias():
        out_ref[...] += c * f2b_ref[0]

    out_ref[...] += c * eo


@jax.jit
def kernel(images, proj_w, proj_b, router_w, ln_g, ln_b,
           fc1_w, fc1_b, fc2_w, fc2_b):
    x6 = images.reshape(B, C, G, P, G, P).transpose(0, 1, 2, 4, 3, 5)
    x6 = jax.lax.optimization_barrier(x6)
    x2 = x6.reshape(B, C, S, PP).transpose(0, 2, 1, 3).reshape(N, K)

    flat, xn, comb = pl.pallas_call(
        _embed_router_body,
        grid=(N // TN,),
        in_specs=[
            pl.BlockSpec((TN, K), lambda n: (n, 0)),
            pl.BlockSpec((D, K), lambda n: (0, 0)),
            pl.BlockSpec((1, D), lambda n: (0, 0)),
            pl.BlockSpec((E, D), lambda n: (0, 0)),
        ],
        out_specs=[
            pl.BlockSpec((TN, D), lambda n: (n, 0)),
            pl.BlockSpec((TN, D), lambda n: (n, 0)),
            pl.BlockSpec((TN, E), lambda n: (n, 0)),
        ],
        out_shape=[
            jax.ShapeDtypeStruct((N, D), jnp.float32),
            jax.ShapeDtypeStruct((N, D), jnp.float32),
            jax.ShapeDtypeStruct((N, E), jnp.float32),
        ],
        scratch_shapes=[pltpu.VMEM((D, K), jnp.bfloat16),
                        pltpu.VMEM((E, K), jnp.bfloat16),
                        pltpu.VMEM((E, K), jnp.bfloat16)],
    )(x2, proj_w, proj_b.reshape(1, D), router_w)

    out = pl.pallas_call(
        _expert_body,
        grid=(E, DFF // TF),
        in_specs=[
            pl.BlockSpec((N, D), lambda e, f: (0, 0)),
            pl.BlockSpec((1, 1, D), lambda e, f: (e, 0, 0)),
            pl.BlockSpec((1, 1, D), lambda e, f: (e, 0, 0)),
            pl.BlockSpec((1, TF, D), lambda e, f: (e, f, 0)),
            pl.BlockSpec((1, 1, TF), lambda e, f: (e, 0, f)),
            pl.BlockSpec((1, D, TF), lambda e, f: (e, 0, f)),
            pl.BlockSpec((1, 1, D), lambda e, f: (e, 0, 0)),
            pl.BlockSpec((N, E), lambda e, f: (0, 0)),
            pl.BlockSpec((N, D), lambda e, f: (0, 0)),
        ],
        out_specs=pl.BlockSpec((N, D), lambda e, f: (0, 0)),
        out_shape=jax.ShapeDtypeStruct((N, D), jnp.float32),
        scratch_shapes=[pltpu.VMEM((N, D), jnp.bfloat16)],
    )(xn, ln_g.reshape(E, 1, D), ln_b.reshape(E, 1, D), fc1_w,
      fc1_b.reshape(E, 1, DFF), fc2_w, fc2_b.reshape(E, 1, D), comb, flat)

    return out.reshape(B, S, D)


# speedup vs baseline: 1.0270x; 1.0112x over previous
import functools

import jax
import jax.numpy as jnp
from jax.experimental import pallas as pl
from jax.experimental.pallas import tpu as pltpu

B = 8
C = 3
IMG = 224
P = 16
D = 768
DFF = 3072
E = 8
G = IMG // P
S = G * G
N = B * S
K = C * P * P
TN = 224
TF = 768


def _embed_router_body(x_ref, pw_ref, pb_ref, rw_ref,
                       flat_ref, xn_ref, comb_ref):
    x = x_ref[...]
    flat = jnp.dot(x, pw_ref[...], preferred_element_type=jnp.float32)
    flat = flat + pb_ref[...]
    flat_ref[...] = flat
    mean = jnp.mean(flat, axis=1, keepdims=True)
    var = jnp.mean((flat - mean) ** 2, axis=1, keepdims=True)
    xn_ref[...] = (flat - mean) * jax.lax.rsqrt(var + 1e-5)

    logits = jnp.dot(flat, rw_ref[...], preferred_element_type=jnp.float32)
    idx = jax.lax.broadcasted_iota(jnp.int32, logits.shape, 1)
    v1 = jnp.max(logits, axis=1, keepdims=True)
    i1 = jnp.min(jnp.where(logits == v1, idx, E), axis=1, keepdims=True)
    rest = jnp.where(idx == i1, -jnp.inf, logits)
    v2 = jnp.max(rest, axis=1, keepdims=True)
    i2 = jnp.min(jnp.where(rest == v2, idx, E), axis=1, keepdims=True)
    w1 = 1.0 / (1.0 + jnp.exp(v2 - v1))
    w2 = 1.0 - w1
    comb_ref[...] = (jnp.where(idx == i1, w1, 0.0)
                     + jnp.where(idx == i2, w2, 0.0))


def _expert_body(xn_ref, lng_ref, lnb_ref, fc1_ref, f1b_ref, fc2_ref,
                 f2b_ref, comb_ref, flat_ref, out_ref, xne_ref):
    e = pl.program_id(0)
    f = pl.program_id(1)

    @pl.when(jnp.logical_and(e == 0, f == 0))
    def _init():
        out_ref[...] = flat_ref[...]

    @pl.when(f == 0)
    def _scale_shift():
        xne_ref[...] = (xn_ref[...] * lng_ref[0]
                        + lnb_ref[0]).astype(jnp.bfloat16)

    eidx = jax.lax.broadcasted_iota(jnp.int32, (N, E), 1)
    c = jnp.sum(jnp.where(eidx == e, comb_ref[...], 0.0), axis=1,
                keepdims=True)

    w1 = fc1_ref[0].astype(jnp.bfloat16)
    h = jax.lax.dot_general(xne_ref[...], w1, (((1,), (1,)), ((), ())),
                            preferred_element_type=jnp.float32) + f1b_ref[0]
    h = jax.nn.gelu(h)
    w2 = fc2_ref[0].astype(jnp.bfloat16)
    eo = jax.lax.dot_general(h.astype(jnp.bfloat16), w2,
                             (((1,), (1,)), ((), ())),
                             preferred_element_type=jnp.float32)

    @pl.when(f == 0)
    def _bias():
        out_ref[...] += c * f2b_ref[0]

    out_ref[...] += c * eo


@jax.jit
def kernel(images, proj_w, proj_b, router_w, ln_g, ln_b,
           fc1_w, fc1_b, fc2_w, fc2_b):
    x = images.reshape(B, C, G, P, G, P).transpose(0, 2, 4, 1, 3, 5)
    x = x.reshape(N, K)

    flat, xn, comb = pl.pallas_call(
        _embed_router_body,
        grid=(N // TN,),
        in_specs=[
            pl.BlockSpec((TN, K), lambda n: (n, 0)),
            pl.BlockSpec((K, D), lambda n: (0, 0)),
            pl.BlockSpec((1, D), lambda n: (0, 0)),
            pl.BlockSpec((D, E), lambda n: (0, 0)),
        ],
        out_specs=[
            pl.BlockSpec((TN, D), lambda n: (n, 0)),
            pl.BlockSpec((TN, D), lambda n: (n, 0)),
            pl.BlockSpec((TN, E), lambda n: (n, 0)),
        ],
        out_shape=[
            jax.ShapeDtypeStruct((N, D), jnp.float32),
            jax.ShapeDtypeStruct((N, D), jnp.float32),
            jax.ShapeDtypeStruct((N, E), jnp.float32),
        ],
    )(x, proj_w.T, proj_b.reshape(1, D), router_w.T)

    out = pl.pallas_call(
        _expert_body,
        grid=(E, DFF // TF),
        in_specs=[
            pl.BlockSpec((N, D), lambda e, f: (0, 0)),
            pl.BlockSpec((1, 1, D), lambda e, f: (e, 0, 0)),
            pl.BlockSpec((1, 1, D), lambda e, f: (e, 0, 0)),
            pl.BlockSpec((1, TF, D), lambda e, f: (e, f, 0)),
            pl.BlockSpec((1, 1, TF), lambda e, f: (e, 0, f)),
            pl.BlockSpec((1, D, TF), lambda e, f: (e, 0, f)),
            pl.BlockSpec((1, 1, D), lambda e, f: (e, 0, 0)),
            pl.BlockSpec((N, E), lambda e, f: (0, 0)),
            pl.BlockSpec((N, D), lambda e, f: (0, 0)),
        ],
        out_specs=pl.BlockSpec((N, D), lambda e, f: (0, 0)),
        out_shape=jax.ShapeDtypeStruct((N, D), jnp.float32),
        scratch_shapes=[pltpu.VMEM((N, D), jnp.bfloat16)],
    )(xn, ln_g.reshape(E, 1, D), ln_b.reshape(E, 1, D), fc1_w,
      fc1_b.reshape(E, 1, DFF), fc2_w, fc2_b.reshape(E, 1, D), comb, flat)

    return out.reshape(B, S, D)
